# Initial kernel scaffold; baseline (speedup 1.0000x reference)
import jax
import jax.numpy as jnp
from jax.experimental import pallas as pl


def kernel(qlen, klen, W):
    raise NotImplementedError("write your pallas kernel here")


# speedup vs baseline: 103.3753x; 103.3753x over previous
import math

import jax
import jax.numpy as jnp
from jax import lax
from jax.experimental import pallas as pl
from jax.experimental.pallas import tpu as pltpu
from jax.experimental.pallas import tpu_sc as plsc

NUM_BUCKETS = 32
MAX_DISTANCE = 128
N_HEADS = 16
QLEN = 2048
KLEN = 2048
DPAD = 4096
TQ = 256
NQB = QLEN // TQ
W2 = TQ + KLEN

_SC_TILES = 32
_TPQ = _SC_TILES // NQB
_HPT = N_HEADS // _TPQ
_VECS = W2 // 16


def bucket_body(idx_ref):
    i = lax.broadcasted_iota(jnp.int32, (1, DPAD), 1)
    relative_position = i - (QLEN - 1)
    n = -relative_position
    half = NUM_BUCKETS // 2
    ret = (n < 0).astype(jnp.int32) * half
    n = jnp.abs(n)
    max_exact = half // 2
    is_small = n < max_exact
    n_safe = jnp.maximum(n, 1)
    val_if_large = max_exact + (
        jnp.log(n_safe.astype(jnp.float32) / max_exact)
        / math.log(MAX_DISTANCE / max_exact)
        * (half - max_exact)
    ).astype(jnp.int32)
    val_if_large = jnp.minimum(val_if_large, jnp.full_like(val_if_large, half - 1))
    idx_ref[...] = ret + jnp.where(is_small, n, val_if_large)


def sc_gather_body(w_hbm, idx_hbm, win_hbm, w_v, idx_v, out_v):
    c = lax.axis_index("c")
    s = lax.axis_index("s")
    wid = s * 2 + c
    qb = wid // _TPQ
    h0 = (wid % _TPQ) * _HPT
    flat0 = (NQB - 1 - qb) * TQ
    pltpu.sync_copy(idx_hbm.at[0, pl.ds(flat0, W2)], idx_v)
    pltpu.sync_copy(w_hbm, w_v)
    for hh in range(_HPT):
        hv = jnp.full((16,), h0 + hh, jnp.int32)
        for j in range(_VECS):
            iv = idx_v[pl.ds(j * 16, 16)]
            out_v[hh, pl.ds(j * 16, 16)] = plsc.load_gather(w_v, (iv, hv))
    for hh in range(_HPT):
        pltpu.sync_copy(out_v.at[hh], win_hbm.at[qb, h0 + hh, 0])


def expand_body(win_ref, out_ref):
    row = win_ref[0, 0, :, :]
    base2 = jnp.broadcast_to(row, (8, W2))
    sub = lax.broadcasted_iota(jnp.int32, (8, W2), 0)
    bank = base2
    for s in range(7):
        shifted = pltpu.roll(base2, W2 - (7 - s), axis=1)
        bank = jnp.where(sub == s, shifted, bank)
    for gg in range(16):
        rho = (TQ - 8 - 8 * gg) % 128
        rolled = pltpu.roll(bank, W2 - rho, axis=1) if rho else bank
        for g in (gg, gg + 16):
            a = (TQ - 8 - 8 * g) - rho
            out_ref[0, 0, pl.ds(8 * g, 8), :] = rolled[:, a:a + KLEN]


def _bucket_call():
    return pl.pallas_call(
        bucket_body,
        out_shape=jax.ShapeDtypeStruct((1, DPAD), jnp.int32),
    )()


def _sc_gather_call(w, idx):
    mesh = plsc.VectorSubcoreMesh(core_axis_name="c", subcore_axis_name="s")
    return pl.kernel(
        sc_gather_body,
        out_type=jax.ShapeDtypeStruct((NQB, N_HEADS, 1, W2), jnp.float32),
        mesh=mesh,
        scratch_types=[
            pltpu.VMEM((NUM_BUCKETS, N_HEADS), jnp.float32),
            pltpu.VMEM((W2,), jnp.int32),
            pltpu.VMEM((_HPT, W2), jnp.float32),
        ],
        compiler_params=pltpu.CompilerParams(needs_layout_passes=False),
    )(w, idx)


def _expand_call(win):
    return pl.pallas_call(
        expand_body,
        grid=(N_HEADS, NQB),
        in_specs=[pl.BlockSpec((1, 1, 1, W2), lambda h, qb: (qb, h, 0, 0))],
        out_specs=pl.BlockSpec((1, 1, TQ, KLEN), lambda h, qb: (0, h, qb, 0)),
        out_shape=jax.ShapeDtypeStruct((1, N_HEADS, QLEN, KLEN), jnp.float32),
    )(win)


def kernel(qlen, klen, W):
    del qlen, klen
    idx = _bucket_call()
    win = _sc_gather_call(W, idx)
    return _expand_call(win)


# speedup vs baseline: 151.1984x; 1.4626x over previous
import math

import jax
import jax.numpy as jnp
from jax import lax
from jax.experimental import pallas as pl
from jax.experimental.pallas import tpu as pltpu
from jax.experimental.pallas import tpu_sc as plsc

NUM_BUCKETS = 32
MAX_DISTANCE = 128
N_HEADS = 16
QLEN = 2048
KLEN = 2048
DPAD = 4096
TQ = 1024
NQB = QLEN // TQ
W2 = TQ + KLEN

_SC_TILES = 32
_TPQ = _SC_TILES // NQB
_HPT = N_HEADS // _TPQ
_VECS = W2 // 16


def bucket_body(idx_ref):
    i = lax.broadcasted_iota(jnp.int32, (1, DPAD), 1)
    relative_position = i - (QLEN - 1)
    n = -relative_position
    half = NUM_BUCKETS // 2
    ret = (n < 0).astype(jnp.int32) * half
    n = jnp.abs(n)
    max_exact = half // 2
    is_small = n < max_exact
    n_safe = jnp.maximum(n, 1)
    val_if_large = max_exact + (
        jnp.log(n_safe.astype(jnp.float32) / max_exact)
        / math.log(MAX_DISTANCE / max_exact)
        * (half - max_exact)
    ).astype(jnp.int32)
    val_if_large = jnp.minimum(val_if_large, jnp.full_like(val_if_large, half - 1))
    idx_ref[...] = ret + jnp.where(is_small, n, val_if_large)


def sc_gather_body(w_hbm, idx_hbm, win_hbm, w_v, idx_v, out_v):
    c = lax.axis_index("c")
    s = lax.axis_index("s")
    wid = s * 2 + c
    qb = wid // _TPQ
    h0 = (wid % _TPQ) * _HPT
    flat0 = (NQB - 1 - qb) * TQ
    pltpu.sync_copy(idx_hbm.at[0, pl.ds(flat0, W2)], idx_v)
    pltpu.sync_copy(w_hbm, w_v)
    for hh in range(_HPT):
        hv = jnp.full((16,), h0 + hh, jnp.int32)
        for j in range(_VECS):
            iv = idx_v[pl.ds(j * 16, 16)]
            out_v[hh, pl.ds(j * 16, 16)] = plsc.load_gather(w_v, (iv, hv))
    for hh in range(_HPT):
        pltpu.sync_copy(out_v.at[hh], win_hbm.at[qb, h0 + hh, 0])


def expand_body(win_ref, out_ref):
    row = win_ref[0, 0, :, :]
    base2 = jnp.broadcast_to(row, (8, W2))
    sub = lax.broadcasted_iota(jnp.int32, (8, W2), 0)
    bank = base2
    for s in range(7):
        shifted = pltpu.roll(base2, W2 - (7 - s), axis=1)
        bank = jnp.where(sub == s, shifted, bank)
    for gg in range(16):
        rho = (TQ - 8 - 8 * gg) % 128
        rolled = pltpu.roll(bank, W2 - rho, axis=1) if rho else bank
        for g in range(gg, TQ // 8, 16):
            a = (TQ - 8 - 8 * g) - rho
            out_ref[0, 0, pl.ds(8 * g, 8), :] = rolled[:, a:a + KLEN]


def _bucket_call():
    return pl.pallas_call(
        bucket_body,
        out_shape=jax.ShapeDtypeStruct((1, DPAD), jnp.int32),
    )()


def _sc_gather_call(w, idx):
    mesh = plsc.VectorSubcoreMesh(core_axis_name="c", subcore_axis_name="s")
    return pl.kernel(
        sc_gather_body,
        out_type=jax.ShapeDtypeStruct((NQB, N_HEADS, 1, W2), jnp.float32),
        mesh=mesh,
        scratch_types=[
            pltpu.VMEM((NUM_BUCKETS, N_HEADS), jnp.float32),
            pltpu.VMEM((W2,), jnp.int32),
            pltpu.VMEM((_HPT, W2), jnp.float32),
        ],
        compiler_params=pltpu.CompilerParams(needs_layout_passes=False),
    )(w, idx)


def _expand_call(win):
    return pl.pallas_call(
        expand_body,
        grid=(N_HEADS, NQB),
        in_specs=[pl.BlockSpec((1, 1, 1, W2), lambda h, qb: (qb, h, 0, 0))],
        out_specs=pl.BlockSpec((1, 1, TQ, KLEN), lambda h, qb: (0, h, qb, 0)),
        out_shape=jax.ShapeDtypeStruct((1, N_HEADS, QLEN, KLEN), jnp.float32),
    )(win)


def kernel(qlen, klen, W):
    del qlen, klen
    idx = _bucket_call()
    win = _sc_gather_call(W, idx)
    return _expand_call(win)


# speedup vs baseline: 153.5326x; 1.0154x over previous
import math

import jax
import jax.numpy as jnp
from jax import lax
from jax.experimental import pallas as pl
from jax.experimental.pallas import tpu as pltpu
from jax.experimental.pallas import tpu_sc as plsc

NUM_BUCKETS = 32
MAX_DISTANCE = 128
N_HEADS = 16
QLEN = 2048
KLEN = 2048
DPAD = 4096
TQ = 1024
NQB = QLEN // TQ
W2 = TQ + KLEN

_SC_TILES = 32
_TPQ = _SC_TILES // NQB
_HPT = N_HEADS // _TPQ
_VECS = W2 // 16


def bucket_body(idx_ref):
    i = lax.broadcasted_iota(jnp.int32, (1, DPAD), 1)
    relative_position = i - (QLEN - 1)
    n = -relative_position
    half = NUM_BUCKETS // 2
    ret = (n < 0).astype(jnp.int32) * half
    n = jnp.abs(n)
    max_exact = half // 2
    is_small = n < max_exact
    n_safe = jnp.maximum(n, 1)
    val_if_large = max_exact + (
        jnp.log(n_safe.astype(jnp.float32) / max_exact)
        / math.log(MAX_DISTANCE / max_exact)
        * (half - max_exact)
    ).astype(jnp.int32)
    val_if_large = jnp.minimum(val_if_large, jnp.full_like(val_if_large, half - 1))
    idx_ref[...] = ret + jnp.where(is_small, n, val_if_large)


_NSAT = 91
_THR = (12, 16, 23, 32, 46, 64, 91)


def _vec_safe(j):
    for qb in range(NQB):
        f0 = (NQB - 1 - qb) * TQ
        p0 = f0 + 16 * j
        if not (p0 + 15 <= (QLEN - 1) - _NSAT or p0 >= (QLEN - 1) + _NSAT):
            return False
    return True


def sc_gather_body(w_hbm, win_hbm, w_v, out_v):
    c = lax.axis_index("c")
    s = lax.axis_index("s")
    wid = s * 2 + c
    qb = wid // _TPQ
    h0 = (wid % _TPQ) * _HPT
    flat0 = (NQB - 1 - qb) * TQ
    pltpu.sync_copy(w_hbm, w_v)
    lane = lax.broadcasted_iota(jnp.int32, (16,), 0)
    for hh in range(_HPT):
        hv = jnp.full((16,), h0 + hh, jnp.int32)
        w15 = plsc.load_gather(w_v, (jnp.full((16,), 15, jnp.int32), hv))
        w31 = plsc.load_gather(w_v, (jnp.full((16,), 31, jnp.int32), hv))
        for j in range(_VECS):
            n = (QLEN - 1) - (flat0 + 16 * j) - lane
            if _vec_safe(j):
                out_v[hh, pl.ds(j * 16, 16)] = jnp.where(n >= 0, w15, w31)
            else:
                ret = jnp.where(n < 0, 16, 0)
                na = jnp.abs(n)
                bl = jnp.full((16,), 8, jnp.int32)
                for t in _THR:
                    bl = bl + (na >= t).astype(jnp.int32)
                b = ret + jnp.where(na < 8, na, bl)
                out_v[hh, pl.ds(j * 16, 16)] = plsc.load_gather(w_v, (b, hv))
    for hh in range(_HPT):
        pltpu.sync_copy(out_v.at[hh], win_hbm.at[qb, h0 + hh, 0])


def expand_body(win_ref, out_ref):
    row = win_ref[0, 0, :, :]
    base2 = jnp.broadcast_to(row, (8, W2))
    sub = lax.broadcasted_iota(jnp.int32, (8, W2), 0)
    bank = base2
    for s in range(7):
        shifted = pltpu.roll(base2, W2 - (7 - s), axis=1)
        bank = jnp.where(sub == s, shifted, bank)
    for gg in range(16):
        rho = (TQ - 8 - 8 * gg) % 128
        rolled = pltpu.roll(bank, W2 - rho, axis=1) if rho else bank
        for g in range(gg, TQ // 8, 16):
            a = (TQ - 8 - 8 * g) - rho
            out_ref[0, 0, pl.ds(8 * g, 8), :] = rolled[:, a:a + KLEN]


def _sc_gather_call(w):
    mesh = plsc.VectorSubcoreMesh(core_axis_name="c", subcore_axis_name="s")
    return pl.kernel(
        sc_gather_body,
        out_type=jax.ShapeDtypeStruct((NQB, N_HEADS, 1, W2), jnp.float32),
        mesh=mesh,
        scratch_types=[
            pltpu.VMEM((NUM_BUCKETS, N_HEADS), jnp.float32),
            pltpu.VMEM((_HPT, W2), jnp.float32),
        ],
        compiler_params=pltpu.CompilerParams(needs_layout_passes=False),
    )(w)


def _expand_call(win):
    return pl.pallas_call(
        expand_body,
        grid=(N_HEADS, NQB),
        in_specs=[pl.BlockSpec((1, 1, 1, W2), lambda h, qb: (qb, h, 0, 0))],
        out_specs=pl.BlockSpec((1, 1, TQ, KLEN), lambda h, qb: (0, h, qb, 0)),
        out_shape=jax.ShapeDtypeStruct((1, N_HEADS, QLEN, KLEN), jnp.float32),
    )(win)


def kernel(qlen, klen, W):
    del qlen, klen
    win = _sc_gather_call(W)
    return _expand_call(win)
